# baseline (speedup 1.0000x reference)
import numpy as np

import jax
import jax.numpy as jnp
from jax.experimental import pallas as pl
from jax.experimental.pallas import tpu as pltpu
from jax.experimental.shard_map import shard_map
from jax.sharding import Mesh, PartitionSpec as P

_INV_T = 20.0
_TILES = (5000, 4000, 2000, 1000, 200, 40, 8)


def _mm_kernel(m_ref, x_ref, o_ref):
    a = m_ref[...] * _INV_T
    o_ref[...] = jax.lax.dot_general(
        a, x_ref[...],
        dimension_numbers=(((1,), (1,)), ((), ())),
        preferred_element_type=jnp.float32)


def _local_matmul(x, memory):
    b, k = x.shape
    n = memory.shape[0]
    tile_r = next((t for t in _TILES if n % t == 0), None)
    if tile_r is None:
        return (memory * _INV_T) @ x.T
    return pl.pallas_call(
        _mm_kernel,
        grid=(n // tile_r,),
        in_specs=[
            pl.BlockSpec((tile_r, k), lambda i: (i, 0)),
            pl.BlockSpec((b, k), lambda i: (0, 0)),
        ],
        out_specs=pl.BlockSpec((tile_r, b), lambda i: (i, 0)),
        out_shape=jax.ShapeDtypeStruct((n, b), jnp.float32),
        compiler_params=pltpu.CompilerParams(
            vmem_limit_bytes=63 * 1024 * 1024,
        ),
    )(memory, x)


def kernel(x, y, memory):
    del y
    n = memory.shape[0]
    devs = jax.devices()
    nd = len(devs)
    if nd > 1 and n % nd == 0:
        mesh = Mesh(np.array(devs), ("d",))
        out_t = shard_map(
            _local_matmul,
            mesh=mesh,
            in_specs=(P(None, None), P("d", None)),
            out_specs=P("d", None),
            check_rep=False,
        )(x, memory)
    else:
        out_t = _local_matmul(x, memory)
    return out_t.T


# speedup vs baseline: 2.4004x; 2.4004x over previous
import jax
import jax.numpy as jnp
from jax.experimental import pallas as pl
from jax.experimental.pallas import tpu as pltpu

_INV_T = 20.0
_TILES = (4000, 5000, 2000, 1000, 200, 40, 8)


def _mm_kernel(m_ref, x_ref, o_ref):
    a = m_ref[...] * _INV_T
    o_ref[...] = jax.lax.dot_general(
        a, x_ref[...],
        dimension_numbers=(((1,), (1,)), ((), ())),
        preferred_element_type=jnp.float32)


def kernel(x, y, memory):
    del y
    b, k = x.shape
    n = memory.shape[0]
    tile_r = next(t for t in _TILES if n % t == 0)
    out_t = pl.pallas_call(
        _mm_kernel,
        grid=(n // tile_r,),
        in_specs=[
            pl.BlockSpec((tile_r, k), lambda i: (i, 0)),
            pl.BlockSpec((b, k), lambda i: (0, 0)),
        ],
        out_specs=pl.BlockSpec((tile_r, b), lambda i: (i, 0)),
        out_shape=jax.ShapeDtypeStruct((n, b), jnp.float32),
        compiler_params=pltpu.CompilerParams(
            vmem_limit_bytes=63 * 1024 * 1024,
        ),
    )(memory, x)
    return out_t.T
